# baseline (speedup 1.0000x reference)
import functools

import jax
import jax.numpy as jnp
from jax.experimental import pallas as pl

NUM_CLASSES_ = 1000
B_ = 4096
L_ = 50
B_BLK = 8


def _onehot_block(t_ref, o_ref):
    t = t_ref[...]
    cls = jax.lax.broadcasted_iota(jnp.int32, (B_BLK, NUM_CLASSES_, L_), 1)
    o_ref[...] = (t[:, None, :] == cls).astype(jnp.int32)


@jax.jit
def kernel(target):
    grid = (B_ // B_BLK,)
    return pl.pallas_call(
        _onehot_block,
        grid=grid,
        in_specs=[pl.BlockSpec((B_BLK, L_), lambda i: (i, 0))],
        out_specs=pl.BlockSpec((B_BLK, NUM_CLASSES_, L_), lambda i: (i, 0, 0)),
        out_shape=jax.ShapeDtypeStruct((B_, NUM_CLASSES_, L_), jnp.int32),
    )(target)
